# baseline (speedup 1.0000x reference)
import functools

import jax
import jax.numpy as jnp
from jax import lax
from jax.experimental import pallas as pl
from jax.experimental.pallas import tpu as pltpu
from jax.experimental.pallas import tpu_sc as plsc

NT, NB, NF = 4096, 16, 5
NS = 2048
SHARP = 1.0

PAD = 16
NSTATE = NS + 1
NCHUNK = (NSTATE + 15) // 16
BUF = PAD + NCHUNK * 16

_Q = (-0.023977755309496203, 0.10149543344558111, -0.2102894641348112,
      0.32529345990077335, -0.49937232766751094, 0.9999918165264949)

_NEG = -1.0e30


def _ctc_body(x_hbm, seqv_hbm, seqlens_hbm, out_hbm, xv, sqv, fa, fb, slv, outv):
    c = lax.axis_index("c")
    s = lax.axis_index("s")

    @pl.when(c == 0)
    def _work():
        b = s
        pltpu.sync_copy(x_hbm.at[b], xv)
        pltpu.sync_copy(seqv_hbm.at[b], sqv)
        pltpu.sync_copy(seqlens_hbm, slv)

        neg = jnp.full((16,), _NEG, jnp.float32)
        for cc in range(BUF // 16):
            fa[pl.ds(cc * 16, 16)] = neg
            fb[pl.ds(cc * 16, 16)] = neg
        fa[pl.ds(PAD, 16)] = jnp.where(lax.iota(jnp.int32, 16) == 0, 0.0, _NEG)

        def one_step(t, src, dst):
            t5 = jnp.full((16,), t * NF, jnp.int32)
            x4 = plsc.load_gather(xv, [t5 + (NF - 1)])

            for cc in range(NCHUNK):
                base = cc * 16
                idx = sqv[pl.ds(base, 16)] + t5
                gx = plsc.load_gather(xv, [idx])
                prev_s = src[pl.ds(base + PAD - 1, 16)]
                prev_a = src[pl.ds(base + PAD, 16)]
                a = gx + prev_s
                bb = x4 + prev_a
                m = jnp.maximum(a, bb)
                d = jnp.minimum(a, bb) - m
                z = jnp.exp(d)
                q = jnp.full((16,), _Q[0], jnp.float32)
                for coef in _Q[1:]:
                    q = q * z + coef
                dst[pl.ds(base + PAD, 16)] = m + z * q

        def two_steps(i, _):
            one_step(2 * i, fa, fb)
            one_step(2 * i + 1, fb, fa)
            return 0

        lax.fori_loop(0, NT // 2, two_steps, 0)

        bidx = jnp.full((16,), b, jnp.int32)
        sl = plsc.load_gather(slv, [bidx])
        fin = plsc.load_gather(fa, [sl + PAD])
        outv[pl.ds(0, 16)] = fin * (-1.0 / (NT * SHARP))
        pltpu.sync_copy(outv, out_hbm.at[b])


@jax.jit
def _ctc_sc(xb, seqv, seqlens):
    mesh = plsc.VectorSubcoreMesh(core_axis_name="c", subcore_axis_name="s",
                                  num_cores=2, num_subcores=16)
    f = pl.kernel(
        _ctc_body,
        out_type=jax.ShapeDtypeStruct((NB, 128), jnp.float32),
        mesh=mesh,
        compiler_params=pltpu.CompilerParams(needs_layout_passes=False),
        scratch_types=[
            pltpu.VMEM((NT * NF,), jnp.float32),
            pltpu.VMEM((NCHUNK * 16,), jnp.int32),
            pltpu.VMEM((BUF,), jnp.float32),
            pltpu.VMEM((BUF,), jnp.float32),
            pltpu.VMEM((16,), jnp.int32),
            pltpu.VMEM((128,), jnp.float32),
        ],
    )
    return f(xb, seqv, seqlens)


def kernel(x, seqs, seqlens):
    nt, nb, nf = x.shape
    assert (nt, nb, nf) == (NT, NB, NF)
    xb = jnp.transpose(x, (1, 0, 2)).reshape(NB, NT * NF)
    seqv = jnp.concatenate(
        [jnp.zeros((NB, 1), jnp.int32), seqs.astype(jnp.int32),
         jnp.zeros((NB, NCHUNK * 16 - 1 - NS), jnp.int32)], axis=1)
    out = _ctc_sc(xb, seqv, seqlens.astype(jnp.int32))
    return out[:, :1]


# speedup vs baseline: 10.0450x; 10.0450x over previous
import functools

import jax
import jax.numpy as jnp
from jax import lax
from jax.experimental import pallas as pl
from jax.experimental.pallas import tpu as pltpu
from jax.experimental.pallas import tpu_sc as plsc

NT, NB, NF = 4096, 16, 5
NS = 2048
SHARP = 1.0

PAD = 16
NSTATE = NS + 1
NCHUNK = (NSTATE + 15) // 16
BUF = PAD + NCHUNK * 16

_Q = (-0.023977755309496203, 0.10149543344558111, -0.2102894641348112,
      0.32529345990077335, -0.49937232766751094, 0.9999918165264949)

_NEG = -1.0e30


def _ctc_body(x_hbm, seqv_hbm, seqlens_hbm, out_hbm, xv, sqv, fa, fb, slv, outv):
    c = lax.axis_index("c")
    s = lax.axis_index("s")

    @pl.when(c == 0)
    def _work():
        b = s
        pltpu.sync_copy(x_hbm.at[b], xv)
        pltpu.sync_copy(seqv_hbm.at[b], sqv)
        pltpu.sync_copy(seqlens_hbm, slv)

        neg = jnp.full((16,), _NEG, jnp.float32)
        for cc in range(BUF // 16):
            fa[pl.ds(cc * 16, 16)] = neg
            fb[pl.ds(cc * 16, 16)] = neg
        fa[pl.ds(PAD, 16)] = jnp.where(lax.iota(jnp.int32, 16) == 0, 0.0, _NEG)

        def one_step(t, src, dst):
            t5 = jnp.full((16,), t * NF, jnp.int32)
            x4 = plsc.load_gather(xv, [t5 + (NF - 1)])

            @plsc.parallel_loop(0, NCHUNK * 16, 16, unroll=4)
            def _chunk(base):
                idx = sqv[pl.ds(base, 16)] + t5
                gx = plsc.load_gather(xv, [idx])
                prev_s = src[pl.ds(base + PAD - 1, 16)]
                prev_a = src[pl.ds(base + PAD, 16)]
                a = gx + prev_s
                bb = x4 + prev_a
                m = jnp.maximum(a, bb)
                d = jnp.minimum(a, bb) - m
                z = jnp.exp(d)
                q = jnp.full((16,), _Q[0], jnp.float32)
                for coef in _Q[1:]:
                    q = q * z + coef
                dst[pl.ds(base + PAD, 16)] = m + z * q

        def two_steps(i, _):
            one_step(2 * i, fa, fb)
            one_step(2 * i + 1, fb, fa)
            return 0

        lax.fori_loop(0, NT // 2, two_steps, 0)

        bidx = jnp.full((16,), b, jnp.int32)
        sl = plsc.load_gather(slv, [bidx])
        fin = plsc.load_gather(fa, [sl + PAD])
        outv[pl.ds(0, 16)] = fin * (-1.0 / (NT * SHARP))
        pltpu.sync_copy(outv, out_hbm.at[b])


@jax.jit
def _ctc_sc(xb, seqv, seqlens):
    mesh = plsc.VectorSubcoreMesh(core_axis_name="c", subcore_axis_name="s",
                                  num_cores=2, num_subcores=16)
    f = pl.kernel(
        _ctc_body,
        out_type=jax.ShapeDtypeStruct((NB, 128), jnp.float32),
        mesh=mesh,
        compiler_params=pltpu.CompilerParams(needs_layout_passes=False),
        scratch_types=[
            pltpu.VMEM((NT * NF,), jnp.float32),
            pltpu.VMEM((NCHUNK * 16,), jnp.int32),
            pltpu.VMEM((BUF,), jnp.float32),
            pltpu.VMEM((BUF,), jnp.float32),
            pltpu.VMEM((16,), jnp.int32),
            pltpu.VMEM((128,), jnp.float32),
        ],
    )
    return f(xb, seqv, seqlens)


def kernel(x, seqs, seqlens):
    nt, nb, nf = x.shape
    assert (nt, nb, nf) == (NT, NB, NF)
    xb = jnp.transpose(x, (1, 0, 2)).reshape(NB, NT * NF)
    seqv = jnp.concatenate(
        [jnp.zeros((NB, 1), jnp.int32), seqs.astype(jnp.int32),
         jnp.zeros((NB, NCHUNK * 16 - 1 - NS), jnp.int32)], axis=1)
    out = _ctc_sc(xb, seqv, seqlens.astype(jnp.int32))
    return out[:, :1]
